# baseline (speedup 1.0000x reference)
import functools

import jax
import jax.numpy as jnp
from jax import lax
from jax.experimental import pallas as pl
from jax.experimental.pallas import tpu as pltpu
from jax.experimental.pallas import tpu_sc as plsc

N = 10000
NP = 10240
E = 320000
NC, NS = 2, 16
NW = NC * NS
K = 128
EW = 10240
E_PAD = EW * NW
CH = EW // K
EW2 = E_PAD // NS
CH2 = EW2 // K
RS = NP // NS
EPS = 1e-5

_mesh = plsc.VectorSubcoreMesh(core_axis_name="c", subcore_axis_name="s")
_sc_params = pltpu.CompilerParams(use_tc_tiling_on_sc=False)


_NB = 4


_G = 2
_RING = 4
_NG = CH2 // _G


_GB = 2


@functools.partial(
    pl.kernel,
    out_type=jax.ShapeDtypeStruct((NC, NP, 64), jnp.float32),
    mesh=_mesh,
    scratch_types=[
        pltpu.VMEM((CH2, K), jnp.int32),
        pltpu.VMEM((CH2, K), jnp.int32),
        pltpu.VMEM((2 * _GB, K, 64), jnp.float32),
        pltpu.VMEM_SHARED((NP, 64), jnp.float32),
        pltpu.SemaphoreType.DMA,
        pltpu.SemaphoreType.DMA,
        pltpu.SemaphoreType.DMA,
        pltpu.SemaphoreType.DMA,
    ],
    compiler_params=_sc_params,
)
def _sc_spmm_half(hp, src, dst, out, src_v, dst_v, bufs, acc,
                  gsa, gsb, ssa, ssb):
  c = lax.axis_index("c")
  s = lax.axis_index("s")
  hph = hp.at[c]

  pltpu.sync_copy(src.at[pl.ds(s * CH2, CH2)], src_v)
  pltpu.sync_copy(dst.at[pl.ds(s * CH2, CH2)], dst_v)
  pltpu.sync_copy(hph.at[pl.ds(s * RS, RS)], acc.at[pl.ds(s * RS, RS)])
  plsc.subcore_barrier()

  def pair(p, carry):
    ia = p * 2 * _GB
    ib = ia + _GB
    ga = [pltpu.async_copy(hph.at[src_v.at[ia + b]], bufs.at[b], gsa)
          for b in range(_GB)]
    gb = [pltpu.async_copy(hph.at[src_v.at[ib + b]], bufs.at[_GB + b], gsb)
          for b in range(_GB)]
    for d in ga:
      d.wait()
    sa = [pltpu.async_copy(bufs.at[b], acc.at[dst_v.at[ia + b]], ssa,
                           add=True) for b in range(_GB)]
    for d in gb:
      d.wait()
    sb = [pltpu.async_copy(bufs.at[_GB + b], acc.at[dst_v.at[ib + b]], ssb,
                           add=True) for b in range(_GB)]
    for d in sa:
      d.wait()
    for d in sb:
      d.wait()
    return carry

  lax.fori_loop(0, CH2 // (2 * _GB), pair, 0)
  plsc.subcore_barrier()
  pltpu.sync_copy(acc.at[pl.ds(s * RS, RS)], out.at[c, pl.ds(s * RS, RS)])


@functools.partial(
    pl.kernel,
    out_type=jax.ShapeDtypeStruct((NC, NP, 16), jnp.float32),
    mesh=_mesh,
    scratch_types=[
        pltpu.VMEM((CH, K), jnp.int32),
        pltpu.VMEM((CH, K), jnp.int32),
        pltpu.VMEM((_NB, K, 16), jnp.float32),
        pltpu.VMEM_SHARED((NP, 16), jnp.float32),
    ] + [pltpu.SemaphoreType.DMA] * (2 * _NB),
    compiler_params=_sc_params,
)
def _sc_spmm16(seed, src, dst, out, src_v, dst_v, bufs, acc, *sems):
  gsems = sems[:_NB]
  ssems = sems[_NB:]
  c = lax.axis_index("c")
  s = lax.axis_index("s")
  wid = s * NC + c
  hph = seed.at[0]

  pltpu.sync_copy(src.at[pl.ds(wid * CH, CH)], src_v)
  pltpu.sync_copy(dst.at[pl.ds(wid * CH, CH)], dst_v)
  pltpu.sync_copy(seed.at[c].at[pl.ds(s * RS, RS)], acc.at[pl.ds(s * RS, RS)])
  plsc.subcore_barrier()

  def group(g, carry):
    i0 = g * _NB
    gd = [pltpu.async_copy(hph.at[src_v.at[i0 + b]], bufs.at[b], gsems[b])
          for b in range(_NB)]
    sd = []
    for b in range(_NB):
      gd[b].wait()
      sd.append(pltpu.async_copy(bufs.at[b], acc.at[dst_v.at[i0 + b]],
                                 ssems[b], add=True))
    for b in range(_NB):
      sd[b].wait()
    return carry

  lax.fori_loop(0, CH // _NB, group, 0)
  plsc.subcore_barrier()
  pltpu.sync_copy(acc.at[pl.ds(s * RS, RS)], out.at[c, pl.ds(s * RS, RS)])


@functools.partial(
    pl.kernel,
    out_type=jax.ShapeDtypeStruct((NC, NP, 16), jnp.float32),
    mesh=_mesh,
    scratch_types=[
        pltpu.VMEM((CH, K), jnp.int32),
        pltpu.VMEM((K, 16), jnp.float32),
        pltpu.VMEM_SHARED((NP, 16), jnp.float32),
        pltpu.SemaphoreType.DMA,
    ],
    compiler_params=_sc_params,
)
def _sc_deg(dst, e0, zrows, out, dst_v, e0_v, acc, ssem):
  c = lax.axis_index("c")
  s = lax.axis_index("s")
  wid = s * NC + c
  pltpu.sync_copy(dst.at[pl.ds(wid * CH, CH)], dst_v)
  pltpu.sync_copy(zrows, acc.at[pl.ds(s * RS, RS)])
  pltpu.sync_copy(e0, e0_v)
  plsc.subcore_barrier()

  def body(i, carry):
    d0 = pltpu.async_copy(e0_v, acc.at[dst_v.at[2 * i]], ssem, add=True)
    d1 = pltpu.async_copy(e0_v, acc.at[dst_v.at[2 * i + 1]], ssem, add=True)
    d0.wait()
    d1.wait()
    return carry

  lax.fori_loop(0, CH // 2, body, 0)
  plsc.subcore_barrier()
  pltpu.sync_copy(acc.at[pl.ds(s * RS, RS)], out.at[c, pl.ds(s * RS, RS)])


_BLK = 512
_GRID = NP // _BLK


def _split_store(hp_ref, y):
  hp_ref[0] = y[:, :64]
  hp_ref[1] = y[:, 64:]


_HP_SPEC = pl.BlockSpec((NC, _BLK, 64), lambda i: (0, i, 0))
_HP_SHAPE = jax.ShapeDtypeStruct((NC, NP, 64), jnp.float32)


def _prep_body(degp_ref, x_ref, w_ref, dis_ref, hp_ref):
  deg = degp_ref[0, :, :1] + degp_ref[1, :, :1] + 1.0
  dis = lax.rsqrt(deg)
  dis_ref[...] = dis
  y = dis * jnp.dot(x_ref[...], w_ref[...],
                    preferred_element_type=jnp.float32,
                    precision=lax.Precision.HIGHEST)
  _split_store(hp_ref, y)


def _tc_prep(degp, x_p, w1):
  return pl.pallas_call(
      _prep_body,
      grid=(_GRID,),
      in_specs=[
          pl.BlockSpec((NC, _BLK, 16), lambda i: (0, i, 0)),
          pl.BlockSpec((_BLK, 128), lambda i: (i, 0)),
          pl.BlockSpec((128, 128), lambda i: (0, 0)),
      ],
      out_specs=[
          pl.BlockSpec((_BLK, 1), lambda i: (i, 0)),
          _HP_SPEC,
      ],
      out_shape=[
          jax.ShapeDtypeStruct((NP, 1), jnp.float32),
          _HP_SHAPE,
      ],
  )(degp, x_p, w1)


def _mid_body(split_out, p_ref, dis_ref, b_ref, g_ref, be_ref, rm_ref, rv_ref,
              w_ref, hp_ref):
  dis = dis_ref[...]
  t = dis * jnp.concatenate([p_ref[0], p_ref[1]], axis=1) + b_ref[...]
  a = g_ref[...] * lax.rsqrt(rv_ref[...] + EPS)
  z = jnp.maximum(a * (t - rm_ref[...]) + be_ref[...], 0.0)
  y = dis * jnp.dot(z, w_ref[...],
                    preferred_element_type=jnp.float32,
                    precision=lax.Precision.HIGHEST)
  if split_out:
    _split_store(hp_ref, y)
  else:
    hp_ref[0] = y
    hp_ref[1] = jnp.zeros_like(y)


def _tc_mid(p, dis, b, g, be, rm, rv, w):
  dn = w.shape[1]
  split_out = dn == 128
  return pl.pallas_call(
      functools.partial(_mid_body, split_out),
      grid=(_GRID,),
      in_specs=[
          pl.BlockSpec((NC, _BLK, 64), lambda i: (0, i, 0)),
          pl.BlockSpec((_BLK, 1), lambda i: (i, 0)),
          pl.BlockSpec((1, 128), lambda i: (0, 0)),
          pl.BlockSpec((1, 128), lambda i: (0, 0)),
          pl.BlockSpec((1, 128), lambda i: (0, 0)),
          pl.BlockSpec((1, 128), lambda i: (0, 0)),
          pl.BlockSpec((1, 128), lambda i: (0, 0)),
          pl.BlockSpec((128, dn), lambda i: (0, 0)),
      ],
      out_specs=_HP_SPEC if split_out else pl.BlockSpec(
          (NC, _BLK, dn), lambda i: (0, i, 0)),
      out_shape=_HP_SHAPE if split_out else jax.ShapeDtypeStruct(
          (NC, NP, dn), jnp.float32),
  )(p, dis, b, g, be, rm, rv, w)


_FBLK = 400
_FGRID = N // _FBLK


def _final_body(p_ref, dis_ref, b_ref, out_ref):
  t = dis_ref[...] * (p_ref[0] + p_ref[1]) + b_ref[...]
  m = jnp.max(t, axis=1, keepdims=True)
  e = jnp.exp(t - m)
  out_ref[...] = t - m - jnp.log(jnp.sum(e, axis=1, keepdims=True))


def _tc_final(p, dis, b):
  return pl.pallas_call(
      _final_body,
      grid=(_FGRID,),
      in_specs=[
          pl.BlockSpec((NC, _FBLK, 16), lambda i: (0, i, 0)),
          pl.BlockSpec((_FBLK, 1), lambda i: (i, 0)),
          pl.BlockSpec((1, 16), lambda i: (0, 0)),
      ],
      out_specs=pl.BlockSpec((_FBLK, 16), lambda i: (i, 0)),
      out_shape=jax.ShapeDtypeStruct((N, 16), jnp.float32),
  )(p, dis, b)


def kernel(x, edge_index, W1, b1, gamma1, beta1, rm1, rv1,
           W2, b2, gamma2, beta2, rm2, rv2, W3, b3):
  src = edge_index[0]
  dst = edge_index[1]
  pad = E_PAD - E
  src_p = jnp.concatenate([src, jnp.zeros((pad,), jnp.int32)]).reshape(E_PAD // K, K)
  dst_p = jnp.concatenate([dst, jnp.full((pad,), N, jnp.int32)]).reshape(E_PAD // K, K)
  x_p = jnp.pad(x, ((0, NP - N), (0, 0)))
  z16 = jnp.zeros((RS, 16), jnp.float32)
  e0 = jnp.zeros((K, 16), jnp.float32).at[:, 0].set(1.0)

  b1r, g1r, be1r = b1.reshape(1, -1), gamma1.reshape(1, -1), beta1.reshape(1, -1)
  rm1r, rv1r = rm1.reshape(1, -1), rv1.reshape(1, -1)
  b2r, g2r, be2r = b2.reshape(1, -1), gamma2.reshape(1, -1), beta2.reshape(1, -1)
  rm2r, rv2r = rm2.reshape(1, -1), rv2.reshape(1, -1)
  b3r = b3.reshape(1, -1)

  degp = _sc_deg(dst_p, e0, z16)
  dis, hp1 = _tc_prep(degp, x_p, W1)
  p1 = _sc_spmm_half(hp1, src_p, dst_p)
  hp2 = _tc_mid(p1, dis, b1r, g1r, be1r, rm1r, rv1r, W2)
  p2 = _sc_spmm_half(hp2, src_p, dst_p)
  hp3 = _tc_mid(p2, dis, b2r, g2r, be2r, rm2r, rv2r, W3)
  p3 = _sc_spmm16(hp3, src_p, dst_p)
  return _tc_final(p3, dis, b3r)


# speedup vs baseline: 1.0005x; 1.0005x over previous
import functools

import jax
import jax.numpy as jnp
from jax import lax
from jax.experimental import pallas as pl
from jax.experimental.pallas import tpu as pltpu
from jax.experimental.pallas import tpu_sc as plsc

N = 10000
NP = 10240
E = 320000
NC, NS = 2, 16
NW = NC * NS
K = 128
EW = 10240
E_PAD = EW * NW
CH = EW // K
EW2 = E_PAD // NS
CH2 = EW2 // K
RS = NP // NS
EPS = 1e-5

_mesh = plsc.VectorSubcoreMesh(core_axis_name="c", subcore_axis_name="s")
_sc_params = pltpu.CompilerParams(use_tc_tiling_on_sc=False)


_NB = 4


_G = 2
_RING = 4
_NG = CH2 // _G


@functools.partial(
    pl.kernel,
    out_type=jax.ShapeDtypeStruct((NC, NP, 64), jnp.float32),
    mesh=_mesh,
    scratch_types=[
        pltpu.VMEM((CH2, K), jnp.int32),
        pltpu.VMEM((CH2, K), jnp.int32),
        pltpu.VMEM((_NB, K, 64), jnp.float32),
        pltpu.VMEM_SHARED((NP, 64), jnp.float32),
    ] + [pltpu.SemaphoreType.DMA] * (2 * _NB),
    compiler_params=_sc_params,
)
def _sc_spmm_half(hp, src, dst, out, src_v, dst_v, bufs, acc, *sems):
  gsems = sems[:_NB]
  ssems = sems[_NB:]
  c = lax.axis_index("c")
  s = lax.axis_index("s")
  hph = hp.at[c]

  pltpu.sync_copy(src.at[pl.ds(s * CH2, CH2)], src_v)
  pltpu.sync_copy(dst.at[pl.ds(s * CH2, CH2)], dst_v)
  pltpu.sync_copy(hph.at[pl.ds(s * RS, RS)], acc.at[pl.ds(s * RS, RS)])
  plsc.subcore_barrier()

  def group(g, carry):
    i0 = g * _NB
    gd = [pltpu.async_copy(hph.at[src_v.at[i0 + b]], bufs.at[b], gsems[b])
          for b in range(_NB)]
    sd = []
    for b in range(_NB):
      gd[b].wait()
      sd.append(pltpu.async_copy(bufs.at[b], acc.at[dst_v.at[i0 + b]],
                                 ssems[b], add=True))
    for d in sd:
      d.wait()
    return carry

  lax.fori_loop(0, CH2 // _NB, group, 0)
  plsc.subcore_barrier()
  pltpu.sync_copy(acc.at[pl.ds(s * RS, RS)], out.at[c, pl.ds(s * RS, RS)])


@functools.partial(
    pl.kernel,
    out_type=jax.ShapeDtypeStruct((NC, NP, 16), jnp.float32),
    mesh=_mesh,
    scratch_types=[
        pltpu.VMEM((CH, K), jnp.int32),
        pltpu.VMEM((CH, K), jnp.int32),
        pltpu.VMEM((_NB, K, 16), jnp.float32),
        pltpu.VMEM_SHARED((NP, 16), jnp.float32),
    ] + [pltpu.SemaphoreType.DMA] * (2 * _NB),
    compiler_params=_sc_params,
)
def _sc_spmm16(seed, src, dst, out, src_v, dst_v, bufs, acc, *sems):
  gsems = sems[:_NB]
  ssems = sems[_NB:]
  c = lax.axis_index("c")
  s = lax.axis_index("s")
  wid = s * NC + c
  hph = seed.at[0]

  pltpu.sync_copy(src.at[pl.ds(wid * CH, CH)], src_v)
  pltpu.sync_copy(dst.at[pl.ds(wid * CH, CH)], dst_v)
  pltpu.sync_copy(seed.at[c].at[pl.ds(s * RS, RS)], acc.at[pl.ds(s * RS, RS)])
  plsc.subcore_barrier()

  def group(g, carry):
    i0 = g * _NB
    gd = [pltpu.async_copy(hph.at[src_v.at[i0 + b]], bufs.at[b], gsems[b])
          for b in range(_NB)]
    sd = []
    for b in range(_NB):
      gd[b].wait()
      sd.append(pltpu.async_copy(bufs.at[b], acc.at[dst_v.at[i0 + b]],
                                 ssems[b], add=True))
    for b in range(_NB):
      sd[b].wait()
    return carry

  lax.fori_loop(0, CH // _NB, group, 0)
  plsc.subcore_barrier()
  pltpu.sync_copy(acc.at[pl.ds(s * RS, RS)], out.at[c, pl.ds(s * RS, RS)])


@functools.partial(
    pl.kernel,
    out_type=jax.ShapeDtypeStruct((NC, NP, 16), jnp.float32),
    mesh=_mesh,
    scratch_types=[
        pltpu.VMEM((CH, K), jnp.int32),
        pltpu.VMEM((K, 16), jnp.float32),
        pltpu.VMEM_SHARED((NP, 16), jnp.float32),
        pltpu.SemaphoreType.DMA,
    ],
    compiler_params=_sc_params,
)
def _sc_deg(dst, e0, zrows, out, dst_v, e0_v, acc, ssem):
  c = lax.axis_index("c")
  s = lax.axis_index("s")
  wid = s * NC + c
  pltpu.sync_copy(dst.at[pl.ds(wid * CH, CH)], dst_v)
  pltpu.sync_copy(zrows, acc.at[pl.ds(s * RS, RS)])
  pltpu.sync_copy(e0, e0_v)
  plsc.subcore_barrier()

  def body(i, carry):
    d0 = pltpu.async_copy(e0_v, acc.at[dst_v.at[2 * i]], ssem, add=True)
    d1 = pltpu.async_copy(e0_v, acc.at[dst_v.at[2 * i + 1]], ssem, add=True)
    d0.wait()
    d1.wait()
    return carry

  lax.fori_loop(0, CH // 2, body, 0)
  plsc.subcore_barrier()
  pltpu.sync_copy(acc.at[pl.ds(s * RS, RS)], out.at[c, pl.ds(s * RS, RS)])


_BLK = 512
_GRID = NP // _BLK


def _split_store(hp_ref, y):
  hp_ref[0] = y[:, :64]
  hp_ref[1] = y[:, 64:]


_HP_SPEC = pl.BlockSpec((NC, _BLK, 64), lambda i: (0, i, 0))
_HP_SHAPE = jax.ShapeDtypeStruct((NC, NP, 64), jnp.float32)


def _prep_body(degp_ref, x_ref, w_ref, dis_ref, hp_ref):
  deg = degp_ref[0, :, :1] + degp_ref[1, :, :1] + 1.0
  dis = lax.rsqrt(deg)
  dis_ref[...] = dis
  y = dis * jnp.dot(x_ref[...], w_ref[...],
                    preferred_element_type=jnp.float32,
                    precision=lax.Precision.HIGHEST)
  _split_store(hp_ref, y)


def _tc_prep(degp, x_p, w1):
  return pl.pallas_call(
      _prep_body,
      grid=(_GRID,),
      in_specs=[
          pl.BlockSpec((NC, _BLK, 16), lambda i: (0, i, 0)),
          pl.BlockSpec((_BLK, 128), lambda i: (i, 0)),
          pl.BlockSpec((128, 128), lambda i: (0, 0)),
      ],
      out_specs=[
          pl.BlockSpec((_BLK, 1), lambda i: (i, 0)),
          _HP_SPEC,
      ],
      out_shape=[
          jax.ShapeDtypeStruct((NP, 1), jnp.float32),
          _HP_SHAPE,
      ],
  )(degp, x_p, w1)


def _mid_body(split_out, p_ref, dis_ref, b_ref, g_ref, be_ref, rm_ref, rv_ref,
              w_ref, hp_ref):
  dis = dis_ref[...]
  t = dis * jnp.concatenate([p_ref[0], p_ref[1]], axis=1) + b_ref[...]
  a = g_ref[...] * lax.rsqrt(rv_ref[...] + EPS)
  z = jnp.maximum(a * (t - rm_ref[...]) + be_ref[...], 0.0)
  y = dis * jnp.dot(z, w_ref[...],
                    preferred_element_type=jnp.float32,
                    precision=lax.Precision.HIGHEST)
  if split_out:
    _split_store(hp_ref, y)
  else:
    hp_ref[0] = y
    hp_ref[1] = jnp.zeros_like(y)


def _tc_mid(p, dis, b, g, be, rm, rv, w):
  dn = w.shape[1]
  split_out = dn == 128
  return pl.pallas_call(
      functools.partial(_mid_body, split_out),
      grid=(_GRID,),
      in_specs=[
          pl.BlockSpec((NC, _BLK, 64), lambda i: (0, i, 0)),
          pl.BlockSpec((_BLK, 1), lambda i: (i, 0)),
          pl.BlockSpec((1, 128), lambda i: (0, 0)),
          pl.BlockSpec((1, 128), lambda i: (0, 0)),
          pl.BlockSpec((1, 128), lambda i: (0, 0)),
          pl.BlockSpec((1, 128), lambda i: (0, 0)),
          pl.BlockSpec((1, 128), lambda i: (0, 0)),
          pl.BlockSpec((128, dn), lambda i: (0, 0)),
      ],
      out_specs=_HP_SPEC if split_out else pl.BlockSpec(
          (NC, _BLK, dn), lambda i: (0, i, 0)),
      out_shape=_HP_SHAPE if split_out else jax.ShapeDtypeStruct(
          (NC, NP, dn), jnp.float32),
  )(p, dis, b, g, be, rm, rv, w)


_FBLK = 400
_FGRID = N // _FBLK


def _final_body(p_ref, dis_ref, b_ref, out_ref):
  t = dis_ref[...] * (p_ref[0] + p_ref[1]) + b_ref[...]
  m = jnp.max(t, axis=1, keepdims=True)
  e = jnp.exp(t - m)
  out_ref[...] = t - m - jnp.log(jnp.sum(e, axis=1, keepdims=True))


def _tc_final(p, dis, b):
  return pl.pallas_call(
      _final_body,
      grid=(_FGRID,),
      in_specs=[
          pl.BlockSpec((NC, _FBLK, 16), lambda i: (0, i, 0)),
          pl.BlockSpec((_FBLK, 1), lambda i: (i, 0)),
          pl.BlockSpec((1, 16), lambda i: (0, 0)),
      ],
      out_specs=pl.BlockSpec((_FBLK, 16), lambda i: (i, 0)),
      out_shape=jax.ShapeDtypeStruct((N, 16), jnp.float32),
  )(p, dis, b)


def kernel(x, edge_index, W1, b1, gamma1, beta1, rm1, rv1,
           W2, b2, gamma2, beta2, rm2, rv2, W3, b3):
  src = edge_index[0]
  dst = edge_index[1]
  pad = E_PAD - E
  src_p = jnp.concatenate([src, jnp.zeros((pad,), jnp.int32)]).reshape(E_PAD // K, K)
  dst_p = jnp.concatenate([dst, jnp.full((pad,), N, jnp.int32)]).reshape(E_PAD // K, K)
  x_p = jnp.pad(x, ((0, NP - N), (0, 0)))
  z16 = jnp.zeros((RS, 16), jnp.float32)
  e0 = jnp.zeros((K, 16), jnp.float32).at[:, 0].set(1.0)

  b1r, g1r, be1r = b1.reshape(1, -1), gamma1.reshape(1, -1), beta1.reshape(1, -1)
  rm1r, rv1r = rm1.reshape(1, -1), rv1.reshape(1, -1)
  b2r, g2r, be2r = b2.reshape(1, -1), gamma2.reshape(1, -1), beta2.reshape(1, -1)
  rm2r, rv2r = rm2.reshape(1, -1), rv2.reshape(1, -1)
  b3r = b3.reshape(1, -1)

  degp = _sc_deg(dst_p, e0, z16)
  dis, hp1 = _tc_prep(degp, x_p, W1)
  p1 = _sc_spmm_half(hp1, src_p, dst_p)
  hp2 = _tc_mid(p1, dis, b1r, g1r, be1r, rm1r, rv1r, W2)
  p2 = _sc_spmm_half(hp2, src_p, dst_p)
  hp3 = _tc_mid(p2, dis, b2r, g2r, be2r, rm2r, rv2r, W3)
  p3 = _sc_spmm16(hp3, src_p, dst_p)
  return _tc_final(p3, dis, b3r)


# speedup vs baseline: 1.1239x; 1.1233x over previous
import functools

import jax
import jax.numpy as jnp
from jax import lax
from jax.experimental import pallas as pl
from jax.experimental.pallas import tpu as pltpu
from jax.experimental.pallas import tpu_sc as plsc

N = 10000
NP = 10240
E = 320000
NC, NS = 2, 16
NW = NC * NS
K = 128
EW = 10240
E_PAD = EW * NW
CH = EW // K
EW2 = E_PAD // NS
CH2 = EW2 // K
RS = NP // NS
EPS = 1e-5

_mesh = plsc.VectorSubcoreMesh(core_axis_name="c", subcore_axis_name="s")
_sc_params = pltpu.CompilerParams(use_tc_tiling_on_sc=False)


_NB = 5
_NB16 = 8


_G = 2
_RING = 4
_NG = CH2 // _G


@functools.partial(
    pl.kernel,
    out_type=jax.ShapeDtypeStruct((NC, NP, 64), jnp.float32),
    mesh=_mesh,
    scratch_types=[
        pltpu.VMEM((CH2, K), jnp.int32),
        pltpu.VMEM((CH2, K), jnp.int32),
        pltpu.VMEM((_NB, K, 64), jnp.float32),
        pltpu.VMEM_SHARED((NP, 64), jnp.float32),
    ] + [pltpu.SemaphoreType.DMA] * (2 * _NB),
    compiler_params=_sc_params,
)
def _sc_spmm_half(hp, src, dst, out, src_v, dst_v, bufs, acc, *sems):
  gsems = sems[:_NB]
  ssems = sems[_NB:]
  c = lax.axis_index("c")
  s = lax.axis_index("s")
  hph = hp.at[c]

  pltpu.sync_copy(src.at[pl.ds(s * CH2, CH2)], src_v)
  pltpu.sync_copy(dst.at[pl.ds(s * CH2, CH2)], dst_v)
  pltpu.sync_copy(hph.at[pl.ds(s * RS, RS)], acc.at[pl.ds(s * RS, RS)])
  plsc.subcore_barrier()

  def group(g, carry):
    i0 = g * _NB
    gd = [pltpu.async_copy(hph.at[src_v.at[i0 + b]], bufs.at[b], gsems[b])
          for b in range(_NB)]
    sd = []
    for b in range(_NB):
      gd[b].wait()
      sd.append(pltpu.async_copy(bufs.at[b], acc.at[dst_v.at[i0 + b]],
                                 ssems[b], add=True))
    for d in sd:
      d.wait()
    return carry

  lax.fori_loop(0, CH2 // _NB, group, 0)
  plsc.subcore_barrier()
  pltpu.sync_copy(acc.at[pl.ds(s * RS, RS)], out.at[c, pl.ds(s * RS, RS)])


@functools.partial(
    pl.kernel,
    out_type=jax.ShapeDtypeStruct((NC, NP, 16), jnp.float32),
    mesh=_mesh,
    scratch_types=[
        pltpu.VMEM((CH, K), jnp.int32),
        pltpu.VMEM((CH, K), jnp.int32),
        pltpu.VMEM((_NB16, K, 16), jnp.float32),
        pltpu.VMEM_SHARED((NP, 16), jnp.float32),
    ] + [pltpu.SemaphoreType.DMA] * (2 * _NB16),
    compiler_params=_sc_params,
)
def _sc_spmm16(seed, src, dst, out, src_v, dst_v, bufs, acc, *sems):
  gsems = sems[:_NB16]
  ssems = sems[_NB16:]
  c = lax.axis_index("c")
  s = lax.axis_index("s")
  wid = s * NC + c
  hph = seed.at[0]

  pltpu.sync_copy(src.at[pl.ds(wid * CH, CH)], src_v)
  pltpu.sync_copy(dst.at[pl.ds(wid * CH, CH)], dst_v)
  pltpu.sync_copy(seed.at[c].at[pl.ds(s * RS, RS)], acc.at[pl.ds(s * RS, RS)])
  plsc.subcore_barrier()

  def group(g, carry):
    i0 = g * _NB16
    gd = [pltpu.async_copy(hph.at[src_v.at[i0 + b]], bufs.at[b], gsems[b])
          for b in range(_NB16)]
    sd = []
    for b in range(_NB16):
      gd[b].wait()
      sd.append(pltpu.async_copy(bufs.at[b], acc.at[dst_v.at[i0 + b]],
                                 ssems[b], add=True))
    for b in range(_NB16):
      sd[b].wait()
    return carry

  lax.fori_loop(0, CH // _NB16, group, 0)
  plsc.subcore_barrier()
  pltpu.sync_copy(acc.at[pl.ds(s * RS, RS)], out.at[c, pl.ds(s * RS, RS)])


@functools.partial(
    pl.kernel,
    out_type=jax.ShapeDtypeStruct((NC, NP, 16), jnp.float32),
    mesh=_mesh,
    scratch_types=[
        pltpu.VMEM((CH, K), jnp.int32),
        pltpu.VMEM((K, 16), jnp.float32),
        pltpu.VMEM_SHARED((NP, 16), jnp.float32),
        pltpu.SemaphoreType.DMA,
    ],
    compiler_params=_sc_params,
)
def _sc_deg(dst, e0, zrows, out, dst_v, e0_v, acc, ssem):
  c = lax.axis_index("c")
  s = lax.axis_index("s")
  wid = s * NC + c
  pltpu.sync_copy(dst.at[pl.ds(wid * CH, CH)], dst_v)
  pltpu.sync_copy(zrows, acc.at[pl.ds(s * RS, RS)])
  pltpu.sync_copy(e0, e0_v)
  plsc.subcore_barrier()

  def body(i, carry):
    d0 = pltpu.async_copy(e0_v, acc.at[dst_v.at[2 * i]], ssem, add=True)
    d1 = pltpu.async_copy(e0_v, acc.at[dst_v.at[2 * i + 1]], ssem, add=True)
    d0.wait()
    d1.wait()
    return carry

  lax.fori_loop(0, CH // 2, body, 0)
  plsc.subcore_barrier()
  pltpu.sync_copy(acc.at[pl.ds(s * RS, RS)], out.at[c, pl.ds(s * RS, RS)])


_BLK = 512
_GRID = NP // _BLK


def _split_store(hp_ref, y):
  hp_ref[0] = y[:, :64]
  hp_ref[1] = y[:, 64:]


_HP_SPEC = pl.BlockSpec((NC, _BLK, 64), lambda i: (0, i, 0))
_HP_SHAPE = jax.ShapeDtypeStruct((NC, NP, 64), jnp.float32)


def _mm_body(x_ref, w_ref, y_ref):
  y_ref[...] = jnp.dot(x_ref[...], w_ref[...],
                       preferred_element_type=jnp.float32,
                       precision=lax.Precision.HIGHEST)


def _tc_mm(x_p, w1):
  return pl.pallas_call(
      _mm_body,
      grid=(_GRID,),
      in_specs=[
          pl.BlockSpec((_BLK, 128), lambda i: (i, 0)),
          pl.BlockSpec((128, 128), lambda i: (0, 0)),
      ],
      out_specs=pl.BlockSpec((_BLK, 128), lambda i: (i, 0)),
      out_shape=jax.ShapeDtypeStruct((NP, 128), jnp.float32),
  )(x_p, w1)


def _scale_body(degp_ref, y_ref, dis_ref, hp_ref):
  deg = degp_ref[0, :, :1] + degp_ref[1, :, :1] + 1.0
  dis = lax.rsqrt(deg)
  dis_ref[...] = dis
  _split_store(hp_ref, dis * y_ref[...])


def _tc_scale(degp, y):
  return pl.pallas_call(
      _scale_body,
      grid=(_GRID,),
      in_specs=[
          pl.BlockSpec((NC, _BLK, 16), lambda i: (0, i, 0)),
          pl.BlockSpec((_BLK, 128), lambda i: (i, 0)),
      ],
      out_specs=[
          pl.BlockSpec((_BLK, 1), lambda i: (i, 0)),
          _HP_SPEC,
      ],
      out_shape=[
          jax.ShapeDtypeStruct((NP, 1), jnp.float32),
          _HP_SHAPE,
      ],
  )(degp, y)


def _mid_body(split_out, p_ref, dis_ref, b_ref, g_ref, be_ref, rm_ref, rv_ref,
              w_ref, hp_ref):
  dis = dis_ref[...]
  t = dis * jnp.concatenate([p_ref[0], p_ref[1]], axis=1) + b_ref[...]
  a = g_ref[...] * lax.rsqrt(rv_ref[...] + EPS)
  z = jnp.maximum(a * (t - rm_ref[...]) + be_ref[...], 0.0)
  y = dis * jnp.dot(z, w_ref[...],
                    preferred_element_type=jnp.float32,
                    precision=lax.Precision.HIGHEST)
  if split_out:
    _split_store(hp_ref, y)
  else:
    hp_ref[0] = y
    hp_ref[1] = jnp.zeros_like(y)


def _tc_mid(p, dis, b, g, be, rm, rv, w):
  dn = w.shape[1]
  split_out = dn == 128
  return pl.pallas_call(
      functools.partial(_mid_body, split_out),
      grid=(_GRID,),
      in_specs=[
          pl.BlockSpec((NC, _BLK, 64), lambda i: (0, i, 0)),
          pl.BlockSpec((_BLK, 1), lambda i: (i, 0)),
          pl.BlockSpec((1, 128), lambda i: (0, 0)),
          pl.BlockSpec((1, 128), lambda i: (0, 0)),
          pl.BlockSpec((1, 128), lambda i: (0, 0)),
          pl.BlockSpec((1, 128), lambda i: (0, 0)),
          pl.BlockSpec((1, 128), lambda i: (0, 0)),
          pl.BlockSpec((128, dn), lambda i: (0, 0)),
      ],
      out_specs=_HP_SPEC if split_out else pl.BlockSpec(
          (NC, _BLK, dn), lambda i: (0, i, 0)),
      out_shape=_HP_SHAPE if split_out else jax.ShapeDtypeStruct(
          (NC, NP, dn), jnp.float32),
  )(p, dis, b, g, be, rm, rv, w)


_FBLK = 400
_FGRID = N // _FBLK


def _final_body(p_ref, dis_ref, b_ref, out_ref):
  t = dis_ref[...] * (p_ref[0] + p_ref[1]) + b_ref[...]
  m = jnp.max(t, axis=1, keepdims=True)
  e = jnp.exp(t - m)
  out_ref[...] = t - m - jnp.log(jnp.sum(e, axis=1, keepdims=True))


def _tc_final(p, dis, b):
  return pl.pallas_call(
      _final_body,
      grid=(_FGRID,),
      in_specs=[
          pl.BlockSpec((NC, _FBLK, 16), lambda i: (0, i, 0)),
          pl.BlockSpec((_FBLK, 1), lambda i: (i, 0)),
          pl.BlockSpec((1, 16), lambda i: (0, 0)),
      ],
      out_specs=pl.BlockSpec((_FBLK, 16), lambda i: (i, 0)),
      out_shape=jax.ShapeDtypeStruct((N, 16), jnp.float32),
  )(p, dis, b)


def kernel(x, edge_index, W1, b1, gamma1, beta1, rm1, rv1,
           W2, b2, gamma2, beta2, rm2, rv2, W3, b3):
  src = edge_index[0]
  dst = edge_index[1]
  pad = E_PAD - E
  src_p = jnp.concatenate([src, jnp.zeros((pad,), jnp.int32)]).reshape(E_PAD // K, K)
  dst_p = jnp.concatenate([dst, jnp.full((pad,), N, jnp.int32)]).reshape(E_PAD // K, K)
  x_p = jnp.pad(x, ((0, NP - N), (0, 0)))
  z16 = jnp.zeros((RS, 16), jnp.float32)
  e0 = jnp.zeros((K, 16), jnp.float32).at[:, 0].set(1.0)

  b1r, g1r, be1r = b1.reshape(1, -1), gamma1.reshape(1, -1), beta1.reshape(1, -1)
  rm1r, rv1r = rm1.reshape(1, -1), rv1.reshape(1, -1)
  b2r, g2r, be2r = b2.reshape(1, -1), gamma2.reshape(1, -1), beta2.reshape(1, -1)
  rm2r, rv2r = rm2.reshape(1, -1), rv2.reshape(1, -1)
  b3r = b3.reshape(1, -1)

  xw1 = _tc_mm(x_p, W1)
  degp = _sc_deg(dst_p, e0, z16)
  dis, hp1 = _tc_scale(degp, xw1)
  p1 = _sc_spmm_half(hp1, src_p, dst_p)
  hp2 = _tc_mid(p1, dis, b1r, g1r, be1r, rm1r, rv1r, W2)
  p2 = _sc_spmm_half(hp2, src_p, dst_p)
  hp3 = _tc_mid(p2, dis, b2r, g2r, be2r, rm2r, rv2r, W3)
  p3 = _sc_spmm16(hp3, src_p, dst_p)
  return _tc_final(p3, dis, b3r)


# speedup vs baseline: 1.1308x; 1.0062x over previous
import functools

import jax
import jax.numpy as jnp
from jax import lax
from jax.experimental import pallas as pl
from jax.experimental.pallas import tpu as pltpu
from jax.experimental.pallas import tpu_sc as plsc

N = 10000
NP = 10240
E = 320000
NC, NS = 2, 16
NW = NC * NS
K = 128
EW = 10240
E_PAD = EW * NW
CH = EW // K
EW2 = E_PAD // NS
CH2 = EW2 // K
RS = NP // NS
EPS = 1e-5

_mesh = plsc.VectorSubcoreMesh(core_axis_name="c", subcore_axis_name="s")
_sc_params = pltpu.CompilerParams(use_tc_tiling_on_sc=False)


_NB = 8
_NB16 = 8


_G = 2
_RING = 4
_NG = CH2 // _G


_CHH = CH2 // 2


@functools.partial(
    pl.kernel,
    out_type=jax.ShapeDtypeStruct((NC, NP, 64), jnp.float32),
    mesh=_mesh,
    scratch_types=[
        pltpu.VMEM((_CHH, K), jnp.int32),
        pltpu.VMEM((_CHH, K), jnp.int32),
        pltpu.VMEM((_NB, K, 64), jnp.float32),
        pltpu.VMEM_SHARED((NP, 64), jnp.float32),
    ] + [pltpu.SemaphoreType.DMA] * (_NB + 1),
    compiler_params=_sc_params,
)
def _sc_spmm_half(hp, src, dst, out, src_v, dst_v, bufs, acc, *sems):
  gsems = sems[:_NB]
  ssem = sems[_NB]
  c = lax.axis_index("c")
  s = lax.axis_index("s")
  hph = hp.at[c]

  pltpu.sync_copy(hph.at[pl.ds(s * RS, RS)], acc.at[pl.ds(s * RS, RS)])
  plsc.subcore_barrier()

  def group(g, carry):
    i0 = g * _NB
    gd = [pltpu.async_copy(hph.at[src_v.at[i0 + b]], bufs.at[b], gsems[b])
          for b in range(_NB)]
    sd = []
    for b in range(_NB):
      gd[b].wait()
      sd.append(pltpu.async_copy(bufs.at[b], acc.at[dst_v.at[i0 + b]],
                                 ssem, add=True))
    for d in sd:
      d.wait()
    return carry

  for h in range(2):
    pltpu.sync_copy(src.at[pl.ds(s * CH2 + h * _CHH, _CHH)], src_v)
    pltpu.sync_copy(dst.at[pl.ds(s * CH2 + h * _CHH, _CHH)], dst_v)
    lax.fori_loop(0, _CHH // _NB, group, 0)

  plsc.subcore_barrier()
  pltpu.sync_copy(acc.at[pl.ds(s * RS, RS)], out.at[c, pl.ds(s * RS, RS)])


@functools.partial(
    pl.kernel,
    out_type=jax.ShapeDtypeStruct((NC, NP, 16), jnp.float32),
    mesh=_mesh,
    scratch_types=[
        pltpu.VMEM((CH, K), jnp.int32),
        pltpu.VMEM((CH, K), jnp.int32),
        pltpu.VMEM((_NB16, K, 16), jnp.float32),
        pltpu.VMEM_SHARED((NP, 16), jnp.float32),
    ] + [pltpu.SemaphoreType.DMA] * (2 * _NB16),
    compiler_params=_sc_params,
)
def _sc_spmm16(seed, src, dst, out, src_v, dst_v, bufs, acc, *sems):
  gsems = sems[:_NB16]
  ssems = sems[_NB16:]
  c = lax.axis_index("c")
  s = lax.axis_index("s")
  wid = s * NC + c
  hph = seed.at[0]

  pltpu.sync_copy(src.at[pl.ds(wid * CH, CH)], src_v)
  pltpu.sync_copy(dst.at[pl.ds(wid * CH, CH)], dst_v)
  pltpu.sync_copy(seed.at[c].at[pl.ds(s * RS, RS)], acc.at[pl.ds(s * RS, RS)])
  plsc.subcore_barrier()

  def group(g, carry):
    i0 = g * _NB16
    gd = [pltpu.async_copy(hph.at[src_v.at[i0 + b]], bufs.at[b], gsems[b])
          for b in range(_NB16)]
    sd = []
    for b in range(_NB16):
      gd[b].wait()
      sd.append(pltpu.async_copy(bufs.at[b], acc.at[dst_v.at[i0 + b]],
                                 ssems[b], add=True))
    for b in range(_NB16):
      sd[b].wait()
    return carry

  lax.fori_loop(0, CH // _NB16, group, 0)
  plsc.subcore_barrier()
  pltpu.sync_copy(acc.at[pl.ds(s * RS, RS)], out.at[c, pl.ds(s * RS, RS)])


@functools.partial(
    pl.kernel,
    out_type=jax.ShapeDtypeStruct((NC, NP, 16), jnp.float32),
    mesh=_mesh,
    scratch_types=[
        pltpu.VMEM((CH, K), jnp.int32),
        pltpu.VMEM((K, 16), jnp.float32),
        pltpu.VMEM_SHARED((NP, 16), jnp.float32),
        pltpu.SemaphoreType.DMA,
    ],
    compiler_params=_sc_params,
)
def _sc_deg(dst, e0, zrows, out, dst_v, e0_v, acc, ssem):
  c = lax.axis_index("c")
  s = lax.axis_index("s")
  wid = s * NC + c
  pltpu.sync_copy(dst.at[pl.ds(wid * CH, CH)], dst_v)
  pltpu.sync_copy(zrows, acc.at[pl.ds(s * RS, RS)])
  pltpu.sync_copy(e0, e0_v)
  plsc.subcore_barrier()

  def body(i, carry):
    d0 = pltpu.async_copy(e0_v, acc.at[dst_v.at[2 * i]], ssem, add=True)
    d1 = pltpu.async_copy(e0_v, acc.at[dst_v.at[2 * i + 1]], ssem, add=True)
    d0.wait()
    d1.wait()
    return carry

  lax.fori_loop(0, CH // 2, body, 0)
  plsc.subcore_barrier()
  pltpu.sync_copy(acc.at[pl.ds(s * RS, RS)], out.at[c, pl.ds(s * RS, RS)])


_BLK = 512
_GRID = NP // _BLK


def _split_store(hp_ref, y):
  hp_ref[0] = y[:, :64]
  hp_ref[1] = y[:, 64:]


_HP_SPEC = pl.BlockSpec((NC, _BLK, 64), lambda i: (0, i, 0))
_HP_SHAPE = jax.ShapeDtypeStruct((NC, NP, 64), jnp.float32)


def _mm_body(x_ref, w_ref, y_ref):
  y_ref[...] = jnp.dot(x_ref[...], w_ref[...],
                       preferred_element_type=jnp.float32,
                       precision=lax.Precision.HIGHEST)


def _tc_mm(x_p, w1):
  return pl.pallas_call(
      _mm_body,
      grid=(_GRID,),
      in_specs=[
          pl.BlockSpec((_BLK, 128), lambda i: (i, 0)),
          pl.BlockSpec((128, 128), lambda i: (0, 0)),
      ],
      out_specs=pl.BlockSpec((_BLK, 128), lambda i: (i, 0)),
      out_shape=jax.ShapeDtypeStruct((NP, 128), jnp.float32),
  )(x_p, w1)


def _scale_body(degp_ref, y_ref, dis_ref, hp_ref):
  deg = degp_ref[0, :, :1] + degp_ref[1, :, :1] + 1.0
  dis = lax.rsqrt(deg)
  dis_ref[...] = dis
  _split_store(hp_ref, dis * y_ref[...])


def _tc_scale(degp, y):
  return pl.pallas_call(
      _scale_body,
      grid=(_GRID,),
      in_specs=[
          pl.BlockSpec((NC, _BLK, 16), lambda i: (0, i, 0)),
          pl.BlockSpec((_BLK, 128), lambda i: (i, 0)),
      ],
      out_specs=[
          pl.BlockSpec((_BLK, 1), lambda i: (i, 0)),
          _HP_SPEC,
      ],
      out_shape=[
          jax.ShapeDtypeStruct((NP, 1), jnp.float32),
          _HP_SHAPE,
      ],
  )(degp, y)


def _mid_body(split_out, p_ref, dis_ref, b_ref, g_ref, be_ref, rm_ref, rv_ref,
              w_ref, hp_ref):
  dis = dis_ref[...]
  t = dis * jnp.concatenate([p_ref[0], p_ref[1]], axis=1) + b_ref[...]
  a = g_ref[...] * lax.rsqrt(rv_ref[...] + EPS)
  z = jnp.maximum(a * (t - rm_ref[...]) + be_ref[...], 0.0)
  y = dis * jnp.dot(z, w_ref[...],
                    preferred_element_type=jnp.float32,
                    precision=lax.Precision.HIGHEST)
  if split_out:
    _split_store(hp_ref, y)
  else:
    hp_ref[0] = y
    hp_ref[1] = jnp.zeros_like(y)


def _tc_mid(p, dis, b, g, be, rm, rv, w):
  dn = w.shape[1]
  split_out = dn == 128
  return pl.pallas_call(
      functools.partial(_mid_body, split_out),
      grid=(_GRID,),
      in_specs=[
          pl.BlockSpec((NC, _BLK, 64), lambda i: (0, i, 0)),
          pl.BlockSpec((_BLK, 1), lambda i: (i, 0)),
          pl.BlockSpec((1, 128), lambda i: (0, 0)),
          pl.BlockSpec((1, 128), lambda i: (0, 0)),
          pl.BlockSpec((1, 128), lambda i: (0, 0)),
          pl.BlockSpec((1, 128), lambda i: (0, 0)),
          pl.BlockSpec((1, 128), lambda i: (0, 0)),
          pl.BlockSpec((128, dn), lambda i: (0, 0)),
      ],
      out_specs=_HP_SPEC if split_out else pl.BlockSpec(
          (NC, _BLK, dn), lambda i: (0, i, 0)),
      out_shape=_HP_SHAPE if split_out else jax.ShapeDtypeStruct(
          (NC, NP, dn), jnp.float32),
  )(p, dis, b, g, be, rm, rv, w)


_FBLK = 400
_FGRID = N // _FBLK


def _final_body(p_ref, dis_ref, b_ref, out_ref):
  t = dis_ref[...] * (p_ref[0] + p_ref[1]) + b_ref[...]
  m = jnp.max(t, axis=1, keepdims=True)
  e = jnp.exp(t - m)
  out_ref[...] = t - m - jnp.log(jnp.sum(e, axis=1, keepdims=True))


def _tc_final(p, dis, b):
  return pl.pallas_call(
      _final_body,
      grid=(_FGRID,),
      in_specs=[
          pl.BlockSpec((NC, _FBLK, 16), lambda i: (0, i, 0)),
          pl.BlockSpec((_FBLK, 1), lambda i: (i, 0)),
          pl.BlockSpec((1, 16), lambda i: (0, 0)),
      ],
      out_specs=pl.BlockSpec((_FBLK, 16), lambda i: (i, 0)),
      out_shape=jax.ShapeDtypeStruct((N, 16), jnp.float32),
  )(p, dis, b)


def kernel(x, edge_index, W1, b1, gamma1, beta1, rm1, rv1,
           W2, b2, gamma2, beta2, rm2, rv2, W3, b3):
  src = edge_index[0]
  dst = edge_index[1]
  pad = E_PAD - E
  src_p = jnp.concatenate([src, jnp.zeros((pad,), jnp.int32)]).reshape(E_PAD // K, K)
  dst_p = jnp.concatenate([dst, jnp.full((pad,), N, jnp.int32)]).reshape(E_PAD // K, K)
  x_p = jnp.pad(x, ((0, NP - N), (0, 0)))
  z16 = jnp.zeros((RS, 16), jnp.float32)
  e0 = jnp.zeros((K, 16), jnp.float32).at[:, 0].set(1.0)

  b1r, g1r, be1r = b1.reshape(1, -1), gamma1.reshape(1, -1), beta1.reshape(1, -1)
  rm1r, rv1r = rm1.reshape(1, -1), rv1.reshape(1, -1)
  b2r, g2r, be2r = b2.reshape(1, -1), gamma2.reshape(1, -1), beta2.reshape(1, -1)
  rm2r, rv2r = rm2.reshape(1, -1), rv2.reshape(1, -1)
  b3r = b3.reshape(1, -1)

  xw1 = _tc_mm(x_p, W1)
  degp = _sc_deg(dst_p, e0, z16)
  dis, hp1 = _tc_scale(degp, xw1)
  p1 = _sc_spmm_half(hp1, src_p, dst_p)
  hp2 = _tc_mid(p1, dis, b1r, g1r, be1r, rm1r, rv1r, W2)
  p2 = _sc_spmm_half(hp2, src_p, dst_p)
  hp3 = _tc_mid(p2, dis, b2r, g2r, be2r, rm2r, rv2r, W3)
  p3 = _sc_spmm16(hp3, src_p, dst_p)
  return _tc_final(p3, dis, b3r)


# speedup vs baseline: 1.1319x; 1.0010x over previous
import functools

import jax
import jax.numpy as jnp
from jax import lax
from jax.experimental import pallas as pl
from jax.experimental.pallas import tpu as pltpu
from jax.experimental.pallas import tpu_sc as plsc

N = 10000
NP = 10240
E = 320000
NC, NS = 2, 16
NW = NC * NS
K = 128
EW = 10240
E_PAD = EW * NW
CH = EW // K
EW2 = E_PAD // NS
CH2 = EW2 // K
RS = NP // NS
EPS = 1e-5

_mesh = plsc.VectorSubcoreMesh(core_axis_name="c", subcore_axis_name="s")
_sc_params = pltpu.CompilerParams(use_tc_tiling_on_sc=False)


_NB = 8
_NB16 = 8


_CHH = CH2 // 2


@functools.partial(
    pl.kernel,
    out_type=jax.ShapeDtypeStruct((NC, NP, 64), jnp.float32),
    mesh=_mesh,
    scratch_types=[
        pltpu.VMEM((_CHH, K), jnp.int32),
        pltpu.VMEM((_CHH, K), jnp.int32),
        pltpu.VMEM((_NB, K, 64), jnp.float32),
        pltpu.VMEM_SHARED((NP, 64), jnp.float32),
    ] + [pltpu.SemaphoreType.DMA] * (_NB + 1),
    compiler_params=_sc_params,
)
def _sc_spmm_half(hp, src, dst, out, src_v, dst_v, bufs, acc, *sems):
  gsems = sems[:_NB]
  ssem = sems[_NB]
  c = lax.axis_index("c")
  s = lax.axis_index("s")
  hph = hp.at[c]

  pltpu.sync_copy(hph.at[pl.ds(s * RS, RS)], acc.at[pl.ds(s * RS, RS)])
  plsc.subcore_barrier()

  def group(g, carry):
    i0 = g * _NB
    gd = [pltpu.async_copy(hph.at[src_v.at[i0 + b]], bufs.at[b], gsems[b])
          for b in range(_NB)]
    sd = []
    for b in range(_NB):
      gd[b].wait()
      sd.append(pltpu.async_copy(bufs.at[b], acc.at[dst_v.at[i0 + b]],
                                 ssem, add=True))
    for d in sd:
      d.wait()
    return carry

  for h in range(2):
    pltpu.sync_copy(src.at[pl.ds(s * CH2 + h * _CHH, _CHH)], src_v)
    pltpu.sync_copy(dst.at[pl.ds(s * CH2 + h * _CHH, _CHH)], dst_v)
    lax.fori_loop(0, _CHH // _NB, group, 0)

  plsc.subcore_barrier()
  pltpu.sync_copy(acc.at[pl.ds(s * RS, RS)], out.at[c, pl.ds(s * RS, RS)])


@functools.partial(
    pl.kernel,
    out_type=jax.ShapeDtypeStruct((NC, NP, 16), jnp.float32),
    mesh=_mesh,
    scratch_types=[
        pltpu.VMEM((CH, K), jnp.int32),
        pltpu.VMEM((CH, K), jnp.int32),
        pltpu.VMEM((_NB16, K, 16), jnp.float32),
        pltpu.VMEM_SHARED((NP, 16), jnp.float32),
    ] + [pltpu.SemaphoreType.DMA] * (2 * _NB16),
    compiler_params=_sc_params,
)
def _sc_spmm16(seed, src, dst, out, src_v, dst_v, bufs, acc, *sems):
  gsems = sems[:_NB16]
  ssems = sems[_NB16:]
  c = lax.axis_index("c")
  s = lax.axis_index("s")
  wid = s * NC + c
  hph = seed.at[0]

  pltpu.sync_copy(src.at[pl.ds(wid * CH, CH)], src_v)
  pltpu.sync_copy(dst.at[pl.ds(wid * CH, CH)], dst_v)
  pltpu.sync_copy(seed.at[c].at[pl.ds(s * RS, RS)], acc.at[pl.ds(s * RS, RS)])
  plsc.subcore_barrier()

  def group(g, carry):
    i0 = g * _NB16
    gd = [pltpu.async_copy(hph.at[src_v.at[i0 + b]], bufs.at[b], gsems[b])
          for b in range(_NB16)]
    sd = []
    for b in range(_NB16):
      gd[b].wait()
      sd.append(pltpu.async_copy(bufs.at[b], acc.at[dst_v.at[i0 + b]],
                                 ssems[b], add=True))
    for b in range(_NB16):
      sd[b].wait()
    return carry

  lax.fori_loop(0, CH // _NB16, group, 0)
  plsc.subcore_barrier()
  pltpu.sync_copy(acc.at[pl.ds(s * RS, RS)], out.at[c, pl.ds(s * RS, RS)])


@functools.partial(
    pl.kernel,
    out_type=jax.ShapeDtypeStruct((NC, NP, 16), jnp.float32),
    mesh=_mesh,
    scratch_types=[
        pltpu.VMEM((CH, K), jnp.int32),
        pltpu.VMEM((K, 16), jnp.float32),
        pltpu.VMEM_SHARED((NP, 16), jnp.float32),
        pltpu.SemaphoreType.DMA,
    ],
    compiler_params=_sc_params,
)
def _sc_deg(dst, e0, zrows, out, dst_v, e0_v, acc, ssem):
  c = lax.axis_index("c")
  s = lax.axis_index("s")
  wid = s * NC + c
  pltpu.sync_copy(dst.at[pl.ds(wid * CH, CH)], dst_v)
  pltpu.sync_copy(zrows, acc.at[pl.ds(s * RS, RS)])
  pltpu.sync_copy(e0, e0_v)
  plsc.subcore_barrier()

  def body(i, carry):
    d0 = pltpu.async_copy(e0_v, acc.at[dst_v.at[2 * i]], ssem, add=True)
    d1 = pltpu.async_copy(e0_v, acc.at[dst_v.at[2 * i + 1]], ssem, add=True)
    d0.wait()
    d1.wait()
    return carry

  lax.fori_loop(0, CH // 2, body, 0)
  plsc.subcore_barrier()
  pltpu.sync_copy(acc.at[pl.ds(s * RS, RS)], out.at[c, pl.ds(s * RS, RS)])


_BLK = 512
_GRID = NP // _BLK


def _split_store(hp_ref, y):
  hp_ref[0] = y[:, :64]
  hp_ref[1] = y[:, 64:]


_HP_SPEC = pl.BlockSpec((NC, _BLK, 64), lambda i: (0, i, 0))
_HP_SHAPE = jax.ShapeDtypeStruct((NC, NP, 64), jnp.float32)


def _mm_body(x_ref, w_ref, y_ref):
  y_ref[...] = jnp.dot(x_ref[...], w_ref[...],
                       preferred_element_type=jnp.float32,
                       precision=lax.Precision.HIGHEST)


def _tc_mm(x_p, w1):
  return pl.pallas_call(
      _mm_body,
      grid=(_GRID,),
      in_specs=[
          pl.BlockSpec((_BLK, 128), lambda i: (i, 0)),
          pl.BlockSpec((128, 128), lambda i: (0, 0)),
      ],
      out_specs=pl.BlockSpec((_BLK, 128), lambda i: (i, 0)),
      out_shape=jax.ShapeDtypeStruct((NP, 128), jnp.float32),
  )(x_p, w1)


def _scale_body(degp_ref, y_ref, dis_ref, hp_ref):
  deg = degp_ref[0, :, :1] + degp_ref[1, :, :1] + 1.0
  dis = lax.rsqrt(deg)
  dis_ref[...] = dis
  _split_store(hp_ref, dis * y_ref[...])


def _tc_scale(degp, y):
  return pl.pallas_call(
      _scale_body,
      grid=(_GRID,),
      in_specs=[
          pl.BlockSpec((NC, _BLK, 16), lambda i: (0, i, 0)),
          pl.BlockSpec((_BLK, 128), lambda i: (i, 0)),
      ],
      out_specs=[
          pl.BlockSpec((_BLK, 1), lambda i: (i, 0)),
          _HP_SPEC,
      ],
      out_shape=[
          jax.ShapeDtypeStruct((NP, 1), jnp.float32),
          _HP_SHAPE,
      ],
  )(degp, y)


def _mid_body(split_out, p_ref, dis_ref, b_ref, g_ref, be_ref, rm_ref, rv_ref,
              w_ref, hp_ref):
  dis = dis_ref[...]
  t = dis * jnp.concatenate([p_ref[0], p_ref[1]], axis=1) + b_ref[...]
  a = g_ref[...] * lax.rsqrt(rv_ref[...] + EPS)
  z = jnp.maximum(a * (t - rm_ref[...]) + be_ref[...], 0.0)
  y = dis * jnp.dot(z, w_ref[...],
                    preferred_element_type=jnp.float32,
                    precision=lax.Precision.HIGHEST)
  if split_out:
    _split_store(hp_ref, y)
  else:
    hp_ref[0] = y
    hp_ref[1] = jnp.zeros_like(y)


def _tc_mid(p, dis, b, g, be, rm, rv, w):
  dn = w.shape[1]
  split_out = dn == 128
  return pl.pallas_call(
      functools.partial(_mid_body, split_out),
      grid=(_GRID,),
      in_specs=[
          pl.BlockSpec((NC, _BLK, 64), lambda i: (0, i, 0)),
          pl.BlockSpec((_BLK, 1), lambda i: (i, 0)),
          pl.BlockSpec((1, 128), lambda i: (0, 0)),
          pl.BlockSpec((1, 128), lambda i: (0, 0)),
          pl.BlockSpec((1, 128), lambda i: (0, 0)),
          pl.BlockSpec((1, 128), lambda i: (0, 0)),
          pl.BlockSpec((1, 128), lambda i: (0, 0)),
          pl.BlockSpec((128, dn), lambda i: (0, 0)),
      ],
      out_specs=_HP_SPEC if split_out else pl.BlockSpec(
          (NC, _BLK, dn), lambda i: (0, i, 0)),
      out_shape=_HP_SHAPE if split_out else jax.ShapeDtypeStruct(
          (NC, NP, dn), jnp.float32),
  )(p, dis, b, g, be, rm, rv, w)


_FBLK = 400
_FGRID = N // _FBLK


def _final_body(p_ref, dis_ref, b_ref, out_ref):
  t = dis_ref[...] * (p_ref[0] + p_ref[1]) + b_ref[...]
  m = jnp.max(t, axis=1, keepdims=True)
  e = jnp.exp(t - m)
  out_ref[...] = t - m - jnp.log(jnp.sum(e, axis=1, keepdims=True))


def _tc_final(p, dis, b):
  return pl.pallas_call(
      _final_body,
      grid=(_FGRID,),
      in_specs=[
          pl.BlockSpec((NC, _FBLK, 16), lambda i: (0, i, 0)),
          pl.BlockSpec((_FBLK, 1), lambda i: (i, 0)),
          pl.BlockSpec((1, 16), lambda i: (0, 0)),
      ],
      out_specs=pl.BlockSpec((_FBLK, 16), lambda i: (i, 0)),
      out_shape=jax.ShapeDtypeStruct((N, 16), jnp.float32),
  )(p, dis, b)


def kernel(x, edge_index, W1, b1, gamma1, beta1, rm1, rv1,
           W2, b2, gamma2, beta2, rm2, rv2, W3, b3):
  src = edge_index[0]
  dst = edge_index[1]
  pad = E_PAD - E
  src_p = jnp.concatenate([src, jnp.zeros((pad,), jnp.int32)]).reshape(E_PAD // K, K)
  dst_p = jnp.concatenate([dst, jnp.full((pad,), N, jnp.int32)]).reshape(E_PAD // K, K)
  x_p = jnp.pad(x, ((0, NP - N), (0, 0)))
  z16 = jnp.zeros((RS, 16), jnp.float32)
  e0 = jnp.zeros((K, 16), jnp.float32).at[:, 0].set(1.0)

  b1r, g1r, be1r = b1.reshape(1, -1), gamma1.reshape(1, -1), beta1.reshape(1, -1)
  rm1r, rv1r = rm1.reshape(1, -1), rv1.reshape(1, -1)
  b2r, g2r, be2r = b2.reshape(1, -1), gamma2.reshape(1, -1), beta2.reshape(1, -1)
  rm2r, rv2r = rm2.reshape(1, -1), rv2.reshape(1, -1)
  b3r = b3.reshape(1, -1)

  xw1 = _tc_mm(x_p, W1)
  degp = _sc_deg(dst_p, e0, z16)
  dis, hp1 = _tc_scale(degp, xw1)
  p1 = _sc_spmm_half(hp1, src_p, dst_p)
  hp2 = _tc_mid(p1, dis, b1r, g1r, be1r, rm1r, rv1r, W2)
  p2 = _sc_spmm_half(hp2, src_p, dst_p)
  hp3 = _tc_mid(p2, dis, b2r, g2r, be2r, rm2r, rv2r, W3)
  p3 = _sc_spmm16(hp3, src_p, dst_p)
  return _tc_final(p3, dis, b3r)


# speedup vs baseline: 1.1319x; 1.0000x over previous
import functools

import jax
import jax.numpy as jnp
from jax import lax
from jax.experimental import pallas as pl
from jax.experimental.pallas import tpu as pltpu
from jax.experimental.pallas import tpu_sc as plsc

N = 10000
NP = 10240
E = 320000
NC, NS = 2, 16
NW = NC * NS
K = 128
EW = 10240
E_PAD = EW * NW
CH = EW // K
EW2 = E_PAD // NS
CH2 = EW2 // K
RS = NP // NS
EPS = 1e-5

_mesh = plsc.VectorSubcoreMesh(core_axis_name="c", subcore_axis_name="s",
                               num_cores=NC, num_subcores=NS)
_sc_params = pltpu.CompilerParams(use_tc_tiling_on_sc=False)


_NB = 8
_NB16 = 8


_CHH = CH2 // 2


@functools.partial(
    pl.kernel,
    out_type=jax.ShapeDtypeStruct((NC, NP, 64), jnp.float32),
    mesh=_mesh,
    scratch_types=[
        pltpu.VMEM((_CHH, K), jnp.int32),
        pltpu.VMEM((_CHH, K), jnp.int32),
        pltpu.VMEM((_NB, K, 64), jnp.float32),
        pltpu.VMEM_SHARED((NP, 64), jnp.float32),
    ] + [pltpu.SemaphoreType.DMA] * (_NB + 1),
    compiler_params=_sc_params,
)
def _sc_spmm_half(hp, src, dst, out, src_v, dst_v, bufs, acc, *sems):
  gsems = sems[:_NB]
  ssem = sems[_NB]
  c = lax.axis_index("c")
  s = lax.axis_index("s")
  hph = hp.at[c]

  pltpu.sync_copy(hph.at[pl.ds(s * RS, RS)], acc.at[pl.ds(s * RS, RS)])
  plsc.subcore_barrier()

  def group(g, carry):
    i0 = g * _NB
    gd = [pltpu.async_copy(hph.at[src_v.at[i0 + b]], bufs.at[b], gsems[b])
          for b in range(_NB)]
    sd = []
    for b in range(_NB):
      gd[b].wait()
      sd.append(pltpu.async_copy(bufs.at[b], acc.at[dst_v.at[i0 + b]],
                                 ssem, add=True))
    for d in sd:
      d.wait()
    return carry

  for h in range(2):
    pltpu.sync_copy(src.at[pl.ds(s * CH2 + h * _CHH, _CHH)], src_v)
    pltpu.sync_copy(dst.at[pl.ds(s * CH2 + h * _CHH, _CHH)], dst_v)
    lax.fori_loop(0, _CHH // _NB, group, 0)

  plsc.subcore_barrier()
  pltpu.sync_copy(acc.at[pl.ds(s * RS, RS)], out.at[c, pl.ds(s * RS, RS)])


@functools.partial(
    pl.kernel,
    out_type=jax.ShapeDtypeStruct((NC, NP, 16), jnp.float32),
    mesh=_mesh,
    scratch_types=[
        pltpu.VMEM((CH, K), jnp.int32),
        pltpu.VMEM((CH, K), jnp.int32),
        pltpu.VMEM((_NB16, K, 16), jnp.float32),
        pltpu.VMEM_SHARED((NP, 16), jnp.float32),
    ] + [pltpu.SemaphoreType.DMA] * (2 * _NB16),
    compiler_params=_sc_params,
)
def _sc_spmm16(seed, src, dst, out, src_v, dst_v, bufs, acc, *sems):
  gsems = sems[:_NB16]
  ssems = sems[_NB16:]
  c = lax.axis_index("c")
  s = lax.axis_index("s")
  wid = s * NC + c
  hph = seed.at[0]

  pltpu.sync_copy(src.at[pl.ds(wid * CH, CH)], src_v)
  pltpu.sync_copy(dst.at[pl.ds(wid * CH, CH)], dst_v)
  pltpu.sync_copy(seed.at[c].at[pl.ds(s * RS, RS)], acc.at[pl.ds(s * RS, RS)])
  plsc.subcore_barrier()

  def group(g, carry):
    i0 = g * _NB16
    gd = [pltpu.async_copy(hph.at[src_v.at[i0 + b]], bufs.at[b], gsems[b])
          for b in range(_NB16)]
    sd = []
    for b in range(_NB16):
      gd[b].wait()
      sd.append(pltpu.async_copy(bufs.at[b], acc.at[dst_v.at[i0 + b]],
                                 ssems[b], add=True))
    for b in range(_NB16):
      sd[b].wait()
    return carry

  lax.fori_loop(0, CH // _NB16, group, 0)
  plsc.subcore_barrier()
  pltpu.sync_copy(acc.at[pl.ds(s * RS, RS)], out.at[c, pl.ds(s * RS, RS)])


@functools.partial(
    pl.kernel,
    out_type=jax.ShapeDtypeStruct((NC, NP, 16), jnp.float32),
    mesh=_mesh,
    scratch_types=[
        pltpu.VMEM((CH, K), jnp.int32),
        pltpu.VMEM((K, 16), jnp.float32),
        pltpu.VMEM_SHARED((NP, 16), jnp.float32),
        pltpu.SemaphoreType.DMA,
    ],
    compiler_params=_sc_params,
)
def _sc_deg(dst, e0, zrows, out, dst_v, e0_v, acc, ssem):
  c = lax.axis_index("c")
  s = lax.axis_index("s")
  wid = s * NC + c
  pltpu.sync_copy(dst.at[pl.ds(wid * CH, CH)], dst_v)
  pltpu.sync_copy(zrows, acc.at[pl.ds(s * RS, RS)])
  pltpu.sync_copy(e0, e0_v)
  plsc.subcore_barrier()

  def body(i, carry):
    d0 = pltpu.async_copy(e0_v, acc.at[dst_v.at[2 * i]], ssem, add=True)
    d1 = pltpu.async_copy(e0_v, acc.at[dst_v.at[2 * i + 1]], ssem, add=True)
    d0.wait()
    d1.wait()
    return carry

  lax.fori_loop(0, CH // 2, body, 0)
  plsc.subcore_barrier()
  pltpu.sync_copy(acc.at[pl.ds(s * RS, RS)], out.at[c, pl.ds(s * RS, RS)])


_BLK = 512
_GRID = NP // _BLK


def _split_store(hp_ref, y):
  hp_ref[0] = y[:, :64]
  hp_ref[1] = y[:, 64:]


_HP_SPEC = pl.BlockSpec((NC, _BLK, 64), lambda i: (0, i, 0))
_HP_SHAPE = jax.ShapeDtypeStruct((NC, NP, 64), jnp.float32)


def _mm_body(x_ref, w_ref, y_ref):
  y_ref[...] = jnp.dot(x_ref[...], w_ref[...],
                       preferred_element_type=jnp.float32,
                       precision=lax.Precision.HIGHEST)


def _tc_mm(x_p, w1):
  return pl.pallas_call(
      _mm_body,
      grid=(_GRID,),
      in_specs=[
          pl.BlockSpec((_BLK, 128), lambda i: (i, 0)),
          pl.BlockSpec((128, 128), lambda i: (0, 0)),
      ],
      out_specs=pl.BlockSpec((_BLK, 128), lambda i: (i, 0)),
      out_shape=jax.ShapeDtypeStruct((NP, 128), jnp.float32),
  )(x_p, w1)


def _scale_body(degp_ref, y_ref, dis_ref, hp_ref):
  deg = degp_ref[0, :, :1] + degp_ref[1, :, :1] + 1.0
  dis = lax.rsqrt(deg)
  dis_ref[...] = dis
  _split_store(hp_ref, dis * y_ref[...])


def _tc_scale(degp, y):
  return pl.pallas_call(
      _scale_body,
      grid=(_GRID,),
      in_specs=[
          pl.BlockSpec((NC, _BLK, 16), lambda i: (0, i, 0)),
          pl.BlockSpec((_BLK, 128), lambda i: (i, 0)),
      ],
      out_specs=[
          pl.BlockSpec((_BLK, 1), lambda i: (i, 0)),
          _HP_SPEC,
      ],
      out_shape=[
          jax.ShapeDtypeStruct((NP, 1), jnp.float32),
          _HP_SHAPE,
      ],
  )(degp, y)


def _mid_body(split_out, p_ref, dis_ref, b_ref, g_ref, be_ref, rm_ref, rv_ref,
              w_ref, hp_ref):
  dis = dis_ref[...]
  t = dis * jnp.concatenate([p_ref[0], p_ref[1]], axis=1) + b_ref[...]
  a = g_ref[...] * lax.rsqrt(rv_ref[...] + EPS)
  z = jnp.maximum(a * (t - rm_ref[...]) + be_ref[...], 0.0)
  y = dis * jnp.dot(z, w_ref[...],
                    preferred_element_type=jnp.float32,
                    precision=lax.Precision.HIGHEST)
  if split_out:
    _split_store(hp_ref, y)
  else:
    hp_ref[0] = y
    hp_ref[1] = jnp.zeros_like(y)


def _tc_mid(p, dis, b, g, be, rm, rv, w):
  dn = w.shape[1]
  split_out = dn == 128
  return pl.pallas_call(
      functools.partial(_mid_body, split_out),
      grid=(_GRID,),
      in_specs=[
          pl.BlockSpec((NC, _BLK, 64), lambda i: (0, i, 0)),
          pl.BlockSpec((_BLK, 1), lambda i: (i, 0)),
          pl.BlockSpec((1, 128), lambda i: (0, 0)),
          pl.BlockSpec((1, 128), lambda i: (0, 0)),
          pl.BlockSpec((1, 128), lambda i: (0, 0)),
          pl.BlockSpec((1, 128), lambda i: (0, 0)),
          pl.BlockSpec((1, 128), lambda i: (0, 0)),
          pl.BlockSpec((128, dn), lambda i: (0, 0)),
      ],
      out_specs=_HP_SPEC if split_out else pl.BlockSpec(
          (NC, _BLK, dn), lambda i: (0, i, 0)),
      out_shape=_HP_SHAPE if split_out else jax.ShapeDtypeStruct(
          (NC, NP, dn), jnp.float32),
  )(p, dis, b, g, be, rm, rv, w)


_FBLK = 400
_FGRID = N // _FBLK


def _final_body(p_ref, dis_ref, b_ref, out_ref):
  t = dis_ref[...] * (p_ref[0] + p_ref[1]) + b_ref[...]
  m = jnp.max(t, axis=1, keepdims=True)
  e = jnp.exp(t - m)
  out_ref[...] = t - m - jnp.log(jnp.sum(e, axis=1, keepdims=True))


def _tc_final(p, dis, b):
  return pl.pallas_call(
      _final_body,
      grid=(_FGRID,),
      in_specs=[
          pl.BlockSpec((NC, _FBLK, 16), lambda i: (0, i, 0)),
          pl.BlockSpec((_FBLK, 1), lambda i: (i, 0)),
          pl.BlockSpec((1, 16), lambda i: (0, 0)),
      ],
      out_specs=pl.BlockSpec((_FBLK, 16), lambda i: (i, 0)),
      out_shape=jax.ShapeDtypeStruct((N, 16), jnp.float32),
  )(p, dis, b)


def kernel(x, edge_index, W1, b1, gamma1, beta1, rm1, rv1,
           W2, b2, gamma2, beta2, rm2, rv2, W3, b3):
  src = edge_index[0]
  dst = edge_index[1]
  pad = E_PAD - E
  src_p = jnp.concatenate([src, jnp.zeros((pad,), jnp.int32)]).reshape(E_PAD // K, K)
  dst_p = jnp.concatenate([dst, jnp.full((pad,), N, jnp.int32)]).reshape(E_PAD // K, K)
  x_p = jnp.pad(x, ((0, NP - N), (0, 0)))
  z16 = jnp.zeros((RS, 16), jnp.float32)
  e0 = jnp.zeros((K, 16), jnp.float32).at[:, 0].set(1.0)

  b1r, g1r, be1r = b1.reshape(1, -1), gamma1.reshape(1, -1), beta1.reshape(1, -1)
  rm1r, rv1r = rm1.reshape(1, -1), rv1.reshape(1, -1)
  b2r, g2r, be2r = b2.reshape(1, -1), gamma2.reshape(1, -1), beta2.reshape(1, -1)
  rm2r, rv2r = rm2.reshape(1, -1), rv2.reshape(1, -1)
  b3r = b3.reshape(1, -1)

  xw1 = _tc_mm(x_p, W1)
  degp = _sc_deg(dst_p, e0, z16)
  dis, hp1 = _tc_scale(degp, xw1)
  p1 = _sc_spmm_half(hp1, src_p, dst_p)
  hp2 = _tc_mid(p1, dis, b1r, g1r, be1r, rm1r, rv1r, W2)
  p2 = _sc_spmm_half(hp2, src_p, dst_p)
  hp3 = _tc_mid(p2, dis, b2r, g2r, be2r, rm2r, rv2r, W3)
  p3 = _sc_spmm16(hp3, src_p, dst_p)
  return _tc_final(p3, dis, b3r)
